# baseline (speedup 1.0000x reference)
import functools

import jax
import jax.numpy as jnp
import numpy as np
from jax import lax
from jax.experimental import pallas as pl
from jax.experimental.pallas import tpu as pltpu
from jax.experimental.pallas import tpu_sc as plsc

B, N = 2, 2048
DP, DM, K = 64, 128, 16
R = B * N
RK = B * N * K
EPS = 1e-5

_INTERPRET = False


def _leaky(x):
    return jnp.where(x >= 0, x, 0.2 * x)


def _dgt(x, w):
    return lax.dot_general(x, w, (((1,), (1,)), ((), ())),
                           precision=lax.Precision.HIGHEST,
                           preferred_element_type=jnp.float32)


def _affine(sums, count, gamma, beta):
    mean = sums[0] / count
    var = sums[1] / count - mean * mean
    scale = gamma / jnp.sqrt(var + EPS)
    shift = beta - mean * scale
    return scale, shift


def _a1_body(feats_ref, pos_ref, w1_ref, g1_ref, b1_ref,
             wq_ref, wk_ref, wv_ref, wd1_ref,
             q_ref, k_ref, v_ref, wp_ref):
    h1 = _dgt(feats_ref[...], w1_ref[...])
    cnt = float(R)
    mean = jnp.sum(h1, axis=0, keepdims=True) / cnt
    var = jnp.sum(h1 * h1, axis=0, keepdims=True) / cnt - mean * mean
    scale = g1_ref[...] / jnp.sqrt(var + EPS)
    shift = b1_ref[...] - mean * scale
    x = _leaky(h1 * scale + shift)
    q_ref[...] = _dgt(x, wq_ref[...])
    k_ref[...] = _dgt(x, wk_ref[...])
    v_ref[...] = _dgt(x, wv_ref[...])
    p = pos_ref[...]
    wp_ref[...] = _dgt(p, wd1_ref[...])


def _stage_a1(feats_rows, pos16, wd1p, p):
    return pl.pallas_call(
        _a1_body,
        out_shape=[jax.ShapeDtypeStruct((R, DM), jnp.float32)] * 4,
        interpret=_INTERPRET,
    )(feats_rows, pos16, p['W1'], p['g1'].reshape(1, DM),
      p['b1'].reshape(1, DM), p['Wq'], p['Wk'], p['Wv'], wd1p)


RB2 = 256


def _a2_body(p16_ref, posb_ref, d_ref):
    p3 = p16_ref[...][:, :3]
    pb3 = posb_ref[0]
    g = lax.dot_general(p3, pb3, (((1,), (0,)), ((), ())),
                        preferred_element_type=jnp.float32)
    sqa = jnp.sum(p3 * p3, axis=1, keepdims=True)
    sqc = jnp.sum(pb3 * pb3, axis=0, keepdims=True)
    d_ref[...] = (sqa + sqc) - 2.0 * g


def _stage_a2(pos16, pos):
    return pl.pallas_call(
        _a2_body,
        grid=(B, N // RB2),
        in_specs=[
            pl.BlockSpec((RB2, 16), lambda b, i: (b * (N // RB2) + i, 0)),
            pl.BlockSpec((1, 3, N), lambda b, i: (b, 0, 0)),
        ],
        out_specs=pl.BlockSpec((RB2, N), lambda b, i: (b * (N // RB2) + i, 0)),
        out_shape=jax.ShapeDtypeStruct((R, N), jnp.float32),
        interpret=_INTERPRET,
    )(pos16, pos)


GCH = 128


def _sc_gather(k_rows, v_rows, wp, idx_flat):
    NW = 32
    per_w = RK // NW
    n_ch = per_w // GCH
    mesh = plsc.VectorSubcoreMesh(core_axis_name="c", subcore_axis_name="s")

    @functools.partial(
        pl.kernel,
        mesh=mesh,
        out_type=[
            jax.ShapeDtypeStruct((RK, DM), jnp.float32),
            jax.ShapeDtypeStruct((RK, DM), jnp.float32),
            jax.ShapeDtypeStruct((RK, DM), jnp.float32),
        ],
        scratch_types=[
            pltpu.VMEM((GCH,), jnp.int32),
            pltpu.VMEM((GCH, DM), jnp.float32),
            pltpu.VMEM((GCH, DM), jnp.float32),
            pltpu.VMEM((GCH, DM), jnp.float32),
            pltpu.SemaphoreType.DMA,
            pltpu.SemaphoreType.DMA,
            pltpu.SemaphoreType.DMA,
        ],
    )
    def gather_kernel(k_hbm, v_hbm, p_hbm, idx_hbm, ko_hbm, vo_hbm, po_hbm,
                      idx_v, kbuf, vbuf, pbuf, sk, sv, sp):
        wid = lax.axis_index("s") * 2 + lax.axis_index("c")
        base = wid * per_w

        def body(c, carry):
            off = base + c * GCH
            pltpu.sync_copy(idx_hbm.at[pl.ds(off, GCH)], idx_v)
            ck = pltpu.async_copy(k_hbm.at[idx_v], kbuf, sk)
            cv = pltpu.async_copy(v_hbm.at[idx_v], vbuf, sv)
            cp = pltpu.async_copy(p_hbm.at[idx_v], pbuf, sp)
            ck.wait()
            pltpu.sync_copy(kbuf, ko_hbm.at[pl.ds(off, GCH)])
            cv.wait()
            pltpu.sync_copy(vbuf, vo_hbm.at[pl.ds(off, GCH)])
            cp.wait()
            pltpu.sync_copy(pbuf, po_hbm.at[pl.ds(off, GCH)])
            return carry

        lax.fori_loop(0, n_ch, body, 0)

    return gather_kernel(k_rows, v_rows, wp, idx_flat)


def _sc_topk(d):
    NW = 32
    rows_w = R // NW
    n_ch = N // 16
    mesh = plsc.VectorSubcoreMesh(core_axis_name="c", subcore_axis_name="s")

    @functools.partial(
        pl.kernel,
        mesh=mesh,
        out_type=jax.ShapeDtypeStruct((RK,), jnp.int32),
        scratch_types=[
            pltpu.VMEM((N,), jnp.float32),
            pltpu.VMEM((N,), jnp.float32),
            pltpu.VMEM((rows_w * K,), jnp.int32),
            pltpu.SemaphoreType.DMA,
            pltpu.SemaphoreType.DMA,
        ],
        compiler_params=pltpu.CompilerParams(needs_layout_passes=False),
    )
    def topk_kernel(d_hbm, out_hbm, dbuf0, dbuf1, obuf, sem0, sem1):
        wid = lax.axis_index("s") * 2 + lax.axis_index("c")
        row0 = wid * rows_w
        boff = (wid // (NW // B)) * N
        iota = lax.iota(jnp.int32, 16)
        inf = jnp.full((16,), jnp.inf, jnp.float32)
        zero = jnp.zeros((16,), jnp.int32)
        pltpu.async_copy(d_hbm.at[row0], dbuf0, sem0)
        pltpu.async_copy(d_hbm.at[row0 + 1], dbuf1, sem1)

        NCHAIN = 4

        def row_topk(dbuf):
            def merge(j, carry):
                out = []
                for c in range(NCHAIN):
                    rk, ri = carry[2 * c], carry[2 * c + 1]
                    off = j * (16 * NCHAIN) + c * 16
                    nk = dbuf[pl.ds(off, 16)]
                    ni = iota + off
                    nk_s, ni_s = plsc.sort_key_val(nk, ni, descending=True)
                    m = nk_s < rk
                    rk2, ri2 = plsc.sort_key_val(
                        jnp.where(m, nk_s, rk), jnp.where(m, ni_s, ri))
                    out += [rk2, ri2]
                return tuple(out)

            carry = lax.fori_loop(0, n_ch // NCHAIN, merge,
                                  (inf, zero) * NCHAIN, unroll=8)
            runs = [(carry[2 * c], carry[2 * c + 1]) for c in range(NCHAIN)]
            while len(runs) > 1:
                nxt = []
                for a in range(0, len(runs), 2):
                    (ak, ai), (bk, bi) = runs[a], runs[a + 1]
                    bk_d, bi_d = plsc.sort_key_val(bk, bi, descending=True)
                    m = bk_d < ak
                    nxt.append(tuple(plsc.sort_key_val(
                        jnp.where(m, bk_d, ak), jnp.where(m, bi_d, ai))))
                runs = nxt
            return runs[0]

        def pair(i, carry):
            r = 2 * i
            pltpu.make_async_copy(d_hbm.at[row0], dbuf0, sem0).wait()
            _, ri0 = row_topk(dbuf0)
            obuf[pl.ds(r * K, 16)] = ri0 + boff

            @pl.when(i < rows_w // 2 - 1)
            def _():
                pltpu.async_copy(d_hbm.at[row0 + r + 2], dbuf0, sem0)

            pltpu.make_async_copy(d_hbm.at[row0], dbuf1, sem1).wait()
            _, ri1 = row_topk(dbuf1)
            obuf[pl.ds((r + 1) * K, 16)] = ri1 + boff

            @pl.when(i < rows_w // 2 - 1)
            def _():
                pltpu.async_copy(d_hbm.at[row0 + r + 3], dbuf1, sem1)

            return carry

        lax.fori_loop(0, rows_w // 2, pair, 0)
        pltpu.sync_copy(obuf, out_hbm.at[pl.ds(row0 * K, rows_w * K)])

    return topk_kernel(d)


RB3 = 2048


def _acc_sums(sums_ref, o, first):
    ps = jnp.stack([jnp.sum(o, axis=0), jnp.sum(o * o, axis=0)])

    @pl.when(first)
    def _():
        sums_ref[...] = ps

    @pl.when(jnp.logical_not(first))
    def _():
        sums_ref[...] += ps


def _t1_block(wp_ref, wpg_ref):
    own = wp_ref[...]
    ob = jnp.broadcast_to(own[:, None, :], (RB3 // K, K, DM)).reshape(RB3, DM)
    return ob - wpg_ref[...]


def _a3_body(wp_ref, wpg_ref, sums_ref):
    i = pl.program_id(0)
    _acc_sums(sums_ref, _t1_block(wp_ref, wpg_ref), i == 0)


def _stage_a3(wp, wpg):
    nb = RK // RB3
    return pl.pallas_call(
        _a3_body,
        grid=(nb,),
        in_specs=[
            pl.BlockSpec((RB3 // K, DM), lambda i: (i, 0)),
            pl.BlockSpec((RB3, DM), lambda i: (i, 0)),
        ],
        out_specs=pl.BlockSpec((2, DM), lambda i: (0, 0)),
        out_shape=jax.ShapeDtypeStruct((2, DM), jnp.float32),
        interpret=_INTERPRET,
    )(wp, wpg)


def _a4_body(wp_ref, wpg_ref, w_ref, s_ref, t_ref, o_ref, sums_ref):
    i = pl.program_id(0)
    x = _leaky(_t1_block(wp_ref, wpg_ref) * s_ref[...] + t_ref[...])
    o = _dgt(x, w_ref[...])
    o_ref[...] = o
    _acc_sums(sums_ref, o, i == 0)


def _stage_a4(wp, wpg, w, scale, shift):
    nb = RK // RB3
    return pl.pallas_call(
        _a4_body,
        grid=(nb,),
        in_specs=[
            pl.BlockSpec((RB3 // K, DM), lambda i: (i, 0)),
            pl.BlockSpec((RB3, DM), lambda i: (i, 0)),
            pl.BlockSpec((DM, DM), lambda i: (0, 0)),
            pl.BlockSpec((1, DM), lambda i: (0, 0)),
            pl.BlockSpec((1, DM), lambda i: (0, 0)),
        ],
        out_specs=[
            pl.BlockSpec((RB3, DM), lambda i: (i, 0)),
            pl.BlockSpec((2, DM), lambda i: (0, 0)),
        ],
        out_shape=[
            jax.ShapeDtypeStruct((RK, DM), jnp.float32),
            jax.ShapeDtypeStruct((2, DM), jnp.float32),
        ],
        interpret=_INTERPRET,
    )(wp, wpg, w, scale.reshape(1, DM), shift.reshape(1, DM))


def _mm_body(x_ref, w_ref, s_ref, t_ref, o_ref, sums_ref):
    i = pl.program_id(0)
    x = _leaky(x_ref[...] * s_ref[...] + t_ref[...])
    o = _dgt(x, w_ref[...])
    o_ref[...] = o
    _acc_sums(sums_ref, o, i == 0)


def _mm_stage(x, w, scale, shift):
    nb = RK // RB3
    return pl.pallas_call(
        _mm_body,
        grid=(nb,),
        in_specs=[
            pl.BlockSpec((RB3, DM), lambda i: (i, 0)),
            pl.BlockSpec((DM, DM), lambda i: (0, 0)),
            pl.BlockSpec((1, DM), lambda i: (0, 0)),
            pl.BlockSpec((1, DM), lambda i: (0, 0)),
        ],
        out_specs=[
            pl.BlockSpec((RB3, DM), lambda i: (i, 0)),
            pl.BlockSpec((2, DM), lambda i: (0, 0)),
        ],
        out_shape=[
            jax.ShapeDtypeStruct((RK, DM), jnp.float32),
            jax.ShapeDtypeStruct((2, DM), jnp.float32),
        ],
        interpret=_INTERPRET,
    )(x, w, scale.reshape(1, DM), shift.reshape(1, DM))


def _a5_body(t2_ref, q_ref, kf_ref, w_ref, s_ref, t_ref,
             pe_ref, t3_ref, sums_ref):
    i = pl.program_id(0)
    pe = _leaky(t2_ref[...] * s_ref[...] + t_ref[...])
    pe_ref[...] = pe
    q = q_ref[...]
    qr = jnp.broadcast_to(q[:, None, :], (RB3 // K, K, DM)).reshape(RB3, DM)
    o = _dgt(qr - kf_ref[...] + pe, w_ref[...])
    t3_ref[...] = o
    _acc_sums(sums_ref, o, i == 0)


def _stage_a5(t2, q_rows, kf, wg1, scale, shift):
    nb = RK // RB3
    return pl.pallas_call(
        _a5_body,
        grid=(nb,),
        in_specs=[
            pl.BlockSpec((RB3, DM), lambda i: (i, 0)),
            pl.BlockSpec((RB3 // K, DM), lambda i: (i, 0)),
            pl.BlockSpec((RB3, DM), lambda i: (i, 0)),
            pl.BlockSpec((DM, DM), lambda i: (0, 0)),
            pl.BlockSpec((1, DM), lambda i: (0, 0)),
            pl.BlockSpec((1, DM), lambda i: (0, 0)),
        ],
        out_specs=[
            pl.BlockSpec((RB3, DM), lambda i: (i, 0)),
            pl.BlockSpec((RB3, DM), lambda i: (i, 0)),
            pl.BlockSpec((2, DM), lambda i: (0, 0)),
        ],
        out_shape=[
            jax.ShapeDtypeStruct((RK, DM), jnp.float32),
            jax.ShapeDtypeStruct((RK, DM), jnp.float32),
            jax.ShapeDtypeStruct((2, DM), jnp.float32),
        ],
        interpret=_INTERPRET,
    )(t2, q_rows, kf, wg1, scale.reshape(1, DM), shift.reshape(1, DM))


def _a7_body(t4_ref, vf_ref, pe_ref, s_ref, t_ref, o_ref):
    a2 = _leaky(t4_ref[...] * s_ref[...] + t_ref[...])
    z = (a2 * (1.0 / np.sqrt(np.float32(N)))).reshape(RB3 // K, K, DM)
    m = jnp.max(z, axis=1, keepdims=True)
    e = jnp.exp(z - m)
    p = e / jnp.sum(e, axis=1, keepdims=True)
    w = (vf_ref[...] + pe_ref[...]).reshape(RB3 // K, K, DM)
    o_ref[...] = jnp.sum(p * w, axis=1)


def _stage_a7(t4, vf, pe, scale, shift):
    nb = RK // RB3
    return pl.pallas_call(
        _a7_body,
        grid=(nb,),
        in_specs=[
            pl.BlockSpec((RB3, DM), lambda i: (i, 0)),
            pl.BlockSpec((RB3, DM), lambda i: (i, 0)),
            pl.BlockSpec((RB3, DM), lambda i: (i, 0)),
            pl.BlockSpec((1, DM), lambda i: (0, 0)),
            pl.BlockSpec((1, DM), lambda i: (0, 0)),
        ],
        out_specs=pl.BlockSpec((RB3 // K, DM), lambda i: (i, 0)),
        out_shape=jax.ShapeDtypeStruct((R, DM), jnp.float32),
        interpret=_INTERPRET,
    )(t4, vf, pe, scale.reshape(1, DM), shift.reshape(1, DM))


def _a8_body(res_ref, w2_ref, o_ref, sums_ref):
    o = _dgt(res_ref[...], w2_ref[...])
    o_ref[...] = o
    _acc_sums(sums_ref, o, True)


def _stage_a8(res, w2):
    return pl.pallas_call(
        _a8_body,
        out_shape=[
            jax.ShapeDtypeStruct((R, DP), jnp.float32),
            jax.ShapeDtypeStruct((2, DP), jnp.float32),
        ],
        interpret=_INTERPRET,
    )(res, w2)


def _a9_body(t5_ref, feats_ref, s_ref, t_ref, o_ref):
    y = _leaky(t5_ref[...] * s_ref[...] + t_ref[...])
    o_ref[0] = y.T + feats_ref[0]


def _stage_a9(t5, feats, scale, shift):
    return pl.pallas_call(
        _a9_body,
        grid=(B,),
        in_specs=[
            pl.BlockSpec((N, DP), lambda b: (b, 0)),
            pl.BlockSpec((1, DP, N), lambda b: (b, 0, 0)),
            pl.BlockSpec((1, DP), lambda b: (0, 0)),
            pl.BlockSpec((1, DP), lambda b: (0, 0)),
        ],
        out_specs=pl.BlockSpec((1, DP, N), lambda b: (b, 0, 0)),
        out_shape=jax.ShapeDtypeStruct((B, DP, N), jnp.float32),
        interpret=_INTERPRET,
    )(t5, feats, scale.reshape(1, DP), shift.reshape(1, DP))


def kernel(feats, pos, params):
    p = params

    feats_rows = jnp.transpose(feats, (0, 2, 1)).reshape(R, DP)
    pos_t = jnp.transpose(pos, (0, 2, 1))
    pos16 = jnp.pad(pos_t, ((0, 0), (0, 0), (0, 13))).reshape(R, 16)
    wd1p = jnp.pad(p['Wd1'], ((0, 0), (0, 13)))

    d = _stage_a2(pos16, pos)
    idx_flat = _sc_topk(d)
    q_rows, k_rows, v_rows, wp = _stage_a1(feats_rows, pos16, wd1p, p)

    kf, vf, wpg = _sc_gather(k_rows, v_rows, wp, idx_flat)

    sums1 = _stage_a3(wp, wpg)
    s1, h1 = _affine(sums1, RK, p['gd1'], p['bd1'])
    t2, sums2 = _stage_a4(wp, wpg, p['Wd2'], s1, h1)
    s2, h2 = _affine(sums2, RK, p['gd2'], p['bd2'])
    pe, t3, sums3 = _stage_a5(t2, q_rows, kf, p['Wg1'], s2, h2)
    s3, h3 = _affine(sums3, RK, p['gg1'], p['bg1'])
    t4, sums4 = _mm_stage(t3, p['Wg2'], s3, h3)
    s4, h4 = _affine(sums4, RK, p['gg2'], p['bg2'])
    res = _stage_a7(t4, vf, pe, s4, h4)
    t5, sums5 = _stage_a8(res, p['W2'])
    s5, h5 = _affine(sums5, R, p['g2'], p['b2'])
    return _stage_a9(t5, feats, s5, h5)


# speedup vs baseline: 1.0322x; 1.0322x over previous
import functools

import jax
import jax.numpy as jnp
import numpy as np
from jax import lax
from jax.experimental import pallas as pl
from jax.experimental.pallas import tpu as pltpu
from jax.experimental.pallas import tpu_sc as plsc

B, N = 2, 2048
DP, DM, K = 64, 128, 16
R = B * N
RK = B * N * K
EPS = 1e-5

_INTERPRET = False


def _leaky(x):
    return jnp.where(x >= 0, x, 0.2 * x)


def _dgt(x, w):
    return lax.dot_general(x, w, (((1,), (1,)), ((), ())),
                           precision=lax.Precision.HIGHEST,
                           preferred_element_type=jnp.float32)


def _affine(sums, count, gamma, beta):
    mean = sums[0] / count
    var = sums[1] / count - mean * mean
    scale = gamma / jnp.sqrt(var + EPS)
    shift = beta - mean * scale
    return scale, shift


def _a1_body(feats_ref, pos_ref, w1_ref, g1_ref, b1_ref,
             wq_ref, wk_ref, wv_ref, wd1_ref,
             q_ref, k_ref, v_ref, wp_ref):
    h1 = _dgt(feats_ref[...], w1_ref[...])
    cnt = float(R)
    mean = jnp.sum(h1, axis=0, keepdims=True) / cnt
    var = jnp.sum(h1 * h1, axis=0, keepdims=True) / cnt - mean * mean
    scale = g1_ref[...] / jnp.sqrt(var + EPS)
    shift = b1_ref[...] - mean * scale
    x = _leaky(h1 * scale + shift)
    q_ref[...] = _dgt(x, wq_ref[...])
    k_ref[...] = _dgt(x, wk_ref[...])
    v_ref[...] = _dgt(x, wv_ref[...])
    p = pos_ref[...]
    wp_ref[...] = _dgt(p, wd1_ref[...])


def _stage_a1(feats_rows, pos16, wd1p, p):
    return pl.pallas_call(
        _a1_body,
        out_shape=[jax.ShapeDtypeStruct((R, DM), jnp.float32)] * 4,
        interpret=_INTERPRET,
    )(feats_rows, pos16, p['W1'], p['g1'].reshape(1, DM),
      p['b1'].reshape(1, DM), p['Wq'], p['Wk'], p['Wv'], wd1p)


RB2 = 256


def _a2_body(p16_ref, posb_ref, d_ref):
    p3 = p16_ref[...][:, :3]
    pb3 = posb_ref[0]
    g = lax.dot_general(p3, pb3, (((1,), (0,)), ((), ())),
                        preferred_element_type=jnp.float32)
    sqa = jnp.sum(p3 * p3, axis=1, keepdims=True)
    sqc = jnp.sum(pb3 * pb3, axis=0, keepdims=True)
    d_ref[...] = (sqa + sqc) - 2.0 * g


def _stage_a2(pos16, pos):
    return pl.pallas_call(
        _a2_body,
        grid=(B, N // RB2),
        in_specs=[
            pl.BlockSpec((RB2, 16), lambda b, i: (b * (N // RB2) + i, 0)),
            pl.BlockSpec((1, 3, N), lambda b, i: (b, 0, 0)),
        ],
        out_specs=pl.BlockSpec((RB2, N), lambda b, i: (b * (N // RB2) + i, 0)),
        out_shape=jax.ShapeDtypeStruct((R, N), jnp.float32),
        interpret=_INTERPRET,
    )(pos16, pos)


GCH = 128


def _sc_gather(k_rows, v_rows, wp, idx_flat):
    NW = 32
    per_w = RK // NW
    n_ch = per_w // GCH
    mesh = plsc.VectorSubcoreMesh(core_axis_name="c", subcore_axis_name="s")

    @functools.partial(
        pl.kernel,
        mesh=mesh,
        out_type=[
            jax.ShapeDtypeStruct((RK, DM), jnp.float32),
            jax.ShapeDtypeStruct((RK, DM), jnp.float32),
            jax.ShapeDtypeStruct((RK, DM), jnp.float32),
        ],
        scratch_types=[
            pltpu.VMEM((GCH,), jnp.int32),
            pltpu.VMEM((GCH,), jnp.int32),
            pltpu.VMEM((GCH, DM), jnp.float32),
            pltpu.VMEM((GCH, DM), jnp.float32),
            pltpu.VMEM((GCH, DM), jnp.float32),
            pltpu.VMEM((GCH, DM), jnp.float32),
            pltpu.VMEM((GCH, DM), jnp.float32),
            pltpu.VMEM((GCH, DM), jnp.float32),
            pltpu.SemaphoreType.DMA,
            pltpu.SemaphoreType.DMA,
        ],
    )
    def gather_kernel(k_hbm, v_hbm, p_hbm, idx_hbm, ko_hbm, vo_hbm, po_hbm,
                      idx0, idx1, kb0, vb0, pb0, kb1, vb1, pb1, s0, s1):
        wid = lax.axis_index("s") * 2 + lax.axis_index("c")
        base = wid * per_w
        sets = ((idx0, kb0, vb0, pb0, s0), (idx1, kb1, vb1, pb1, s1))

        def issue(c, st):
            idx_v, kb, vb, pb, sem = st
            off = base + c * GCH
            pltpu.sync_copy(idx_hbm.at[pl.ds(off, GCH)], idx_v)
            pltpu.async_copy(k_hbm.at[idx_v], kb, sem)
            pltpu.async_copy(v_hbm.at[idx_v], vb, sem)
            pltpu.async_copy(p_hbm.at[idx_v], pb, sem)

        def drain(c, st):
            idx_v, kb, vb, pb, sem = st
            off = base + c * GCH
            pltpu.make_async_copy(k_hbm.at[idx_v], kb, sem).wait()
            pltpu.make_async_copy(v_hbm.at[idx_v], vb, sem).wait()
            pltpu.make_async_copy(p_hbm.at[idx_v], pb, sem).wait()
            pltpu.sync_copy(kb, ko_hbm.at[pl.ds(off, GCH)])
            pltpu.sync_copy(vb, vo_hbm.at[pl.ds(off, GCH)])
            pltpu.sync_copy(pb, po_hbm.at[pl.ds(off, GCH)])

        issue(0, sets[0])
        issue(1, sets[1])

        def body(i, carry):
            c = 2 * i
            drain(c, sets[0])

            @pl.when(c + 2 < n_ch)
            def _():
                issue(c + 2, sets[0])

            drain(c + 1, sets[1])

            @pl.when(c + 3 < n_ch)
            def _():
                issue(c + 3, sets[1])

            return carry

        lax.fori_loop(0, n_ch // 2, body, 0)

    return gather_kernel(k_rows, v_rows, wp, idx_flat)


def _sc_topk(d):
    NW = 32
    rows_w = R // NW
    n_ch = N // 16
    mesh = plsc.VectorSubcoreMesh(core_axis_name="c", subcore_axis_name="s")

    @functools.partial(
        pl.kernel,
        mesh=mesh,
        out_type=jax.ShapeDtypeStruct((RK,), jnp.int32),
        scratch_types=[
            pltpu.VMEM((N,), jnp.float32),
            pltpu.VMEM((N,), jnp.float32),
            pltpu.VMEM((rows_w * K,), jnp.int32),
            pltpu.SemaphoreType.DMA,
            pltpu.SemaphoreType.DMA,
        ],
        compiler_params=pltpu.CompilerParams(needs_layout_passes=False),
    )
    def topk_kernel(d_hbm, out_hbm, dbuf0, dbuf1, obuf, sem0, sem1):
        wid = lax.axis_index("s") * 2 + lax.axis_index("c")
        row0 = wid * rows_w
        boff = (wid // (NW // B)) * N
        iota = lax.iota(jnp.int32, 16)
        inf = jnp.full((16,), jnp.inf, jnp.float32)
        zero = jnp.zeros((16,), jnp.int32)
        pltpu.async_copy(d_hbm.at[row0], dbuf0, sem0)
        pltpu.async_copy(d_hbm.at[row0 + 1], dbuf1, sem1)

        NCHAIN = 4

        def row_topk(dbuf):
            def merge(j, carry):
                out = []
                for c in range(NCHAIN):
                    rk, ri = carry[2 * c], carry[2 * c + 1]
                    off = j * (16 * NCHAIN) + c * 16
                    nk = dbuf[pl.ds(off, 16)]
                    ni = iota + off
                    nk_s, ni_s = plsc.sort_key_val(nk, ni, descending=True)
                    m = nk_s < rk
                    rk2, ri2 = plsc.sort_key_val(
                        jnp.where(m, nk_s, rk), jnp.where(m, ni_s, ri))
                    out += [rk2, ri2]
                return tuple(out)

            carry = lax.fori_loop(0, n_ch // NCHAIN, merge,
                                  (inf, zero) * NCHAIN, unroll=8)
            runs = [(carry[2 * c], carry[2 * c + 1]) for c in range(NCHAIN)]
            while len(runs) > 1:
                nxt = []
                for a in range(0, len(runs), 2):
                    (ak, ai), (bk, bi) = runs[a], runs[a + 1]
                    bk_d, bi_d = plsc.sort_key_val(bk, bi, descending=True)
                    m = bk_d < ak
                    nxt.append(tuple(plsc.sort_key_val(
                        jnp.where(m, bk_d, ak), jnp.where(m, bi_d, ai))))
                runs = nxt
            return runs[0]

        def pair(i, carry):
            r = 2 * i
            pltpu.make_async_copy(d_hbm.at[row0], dbuf0, sem0).wait()
            _, ri0 = row_topk(dbuf0)
            obuf[pl.ds(r * K, 16)] = ri0 + boff

            @pl.when(i < rows_w // 2 - 1)
            def _():
                pltpu.async_copy(d_hbm.at[row0 + r + 2], dbuf0, sem0)

            pltpu.make_async_copy(d_hbm.at[row0], dbuf1, sem1).wait()
            _, ri1 = row_topk(dbuf1)
            obuf[pl.ds((r + 1) * K, 16)] = ri1 + boff

            @pl.when(i < rows_w // 2 - 1)
            def _():
                pltpu.async_copy(d_hbm.at[row0 + r + 3], dbuf1, sem1)

            return carry

        lax.fori_loop(0, rows_w // 2, pair, 0)
        pltpu.sync_copy(obuf, out_hbm.at[pl.ds(row0 * K, rows_w * K)])

    return topk_kernel(d)


RB3 = 2048


def _acc_sums(sums_ref, o, first):
    ps = jnp.stack([jnp.sum(o, axis=0), jnp.sum(o * o, axis=0)])

    @pl.when(first)
    def _():
        sums_ref[...] = ps

    @pl.when(jnp.logical_not(first))
    def _():
        sums_ref[...] += ps


def _t1_block(wp_ref, wpg_ref):
    own = wp_ref[...]
    ob = jnp.broadcast_to(own[:, None, :], (RB3 // K, K, DM)).reshape(RB3, DM)
    return ob - wpg_ref[...]


def _a3_body(wp_ref, wpg_ref, sums_ref):
    i = pl.program_id(0)
    _acc_sums(sums_ref, _t1_block(wp_ref, wpg_ref), i == 0)


def _stage_a3(wp, wpg):
    nb = RK // RB3
    return pl.pallas_call(
        _a3_body,
        grid=(nb,),
        in_specs=[
            pl.BlockSpec((RB3 // K, DM), lambda i: (i, 0)),
            pl.BlockSpec((RB3, DM), lambda i: (i, 0)),
        ],
        out_specs=pl.BlockSpec((2, DM), lambda i: (0, 0)),
        out_shape=jax.ShapeDtypeStruct((2, DM), jnp.float32),
        interpret=_INTERPRET,
    )(wp, wpg)


def _a4_body(wp_ref, wpg_ref, w_ref, s_ref, t_ref, o_ref, sums_ref):
    i = pl.program_id(0)
    x = _leaky(_t1_block(wp_ref, wpg_ref) * s_ref[...] + t_ref[...])
    o = _dgt(x, w_ref[...])
    o_ref[...] = o
    _acc_sums(sums_ref, o, i == 0)


def _stage_a4(wp, wpg, w, scale, shift):
    nb = RK // RB3
    return pl.pallas_call(
        _a4_body,
        grid=(nb,),
        in_specs=[
            pl.BlockSpec((RB3 // K, DM), lambda i: (i, 0)),
            pl.BlockSpec((RB3, DM), lambda i: (i, 0)),
            pl.BlockSpec((DM, DM), lambda i: (0, 0)),
            pl.BlockSpec((1, DM), lambda i: (0, 0)),
            pl.BlockSpec((1, DM), lambda i: (0, 0)),
        ],
        out_specs=[
            pl.BlockSpec((RB3, DM), lambda i: (i, 0)),
            pl.BlockSpec((2, DM), lambda i: (0, 0)),
        ],
        out_shape=[
            jax.ShapeDtypeStruct((RK, DM), jnp.float32),
            jax.ShapeDtypeStruct((2, DM), jnp.float32),
        ],
        interpret=_INTERPRET,
    )(wp, wpg, w, scale.reshape(1, DM), shift.reshape(1, DM))


def _mm_body(x_ref, w_ref, s_ref, t_ref, o_ref, sums_ref):
    i = pl.program_id(0)
    x = _leaky(x_ref[...] * s_ref[...] + t_ref[...])
    o = _dgt(x, w_ref[...])
    o_ref[...] = o
    _acc_sums(sums_ref, o, i == 0)


def _mm_stage(x, w, scale, shift):
    nb = RK // RB3
    return pl.pallas_call(
        _mm_body,
        grid=(nb,),
        in_specs=[
            pl.BlockSpec((RB3, DM), lambda i: (i, 0)),
            pl.BlockSpec((DM, DM), lambda i: (0, 0)),
            pl.BlockSpec((1, DM), lambda i: (0, 0)),
            pl.BlockSpec((1, DM), lambda i: (0, 0)),
        ],
        out_specs=[
            pl.BlockSpec((RB3, DM), lambda i: (i, 0)),
            pl.BlockSpec((2, DM), lambda i: (0, 0)),
        ],
        out_shape=[
            jax.ShapeDtypeStruct((RK, DM), jnp.float32),
            jax.ShapeDtypeStruct((2, DM), jnp.float32),
        ],
        interpret=_INTERPRET,
    )(x, w, scale.reshape(1, DM), shift.reshape(1, DM))


def _a5_body(t2_ref, q_ref, kf_ref, w_ref, s_ref, t_ref,
             pe_ref, t3_ref, sums_ref):
    i = pl.program_id(0)
    pe = _leaky(t2_ref[...] * s_ref[...] + t_ref[...])
    pe_ref[...] = pe
    q = q_ref[...]
    qr = jnp.broadcast_to(q[:, None, :], (RB3 // K, K, DM)).reshape(RB3, DM)
    o = _dgt(qr - kf_ref[...] + pe, w_ref[...])
    t3_ref[...] = o
    _acc_sums(sums_ref, o, i == 0)


def _stage_a5(t2, q_rows, kf, wg1, scale, shift):
    nb = RK // RB3
    return pl.pallas_call(
        _a5_body,
        grid=(nb,),
        in_specs=[
            pl.BlockSpec((RB3, DM), lambda i: (i, 0)),
            pl.BlockSpec((RB3 // K, DM), lambda i: (i, 0)),
            pl.BlockSpec((RB3, DM), lambda i: (i, 0)),
            pl.BlockSpec((DM, DM), lambda i: (0, 0)),
            pl.BlockSpec((1, DM), lambda i: (0, 0)),
            pl.BlockSpec((1, DM), lambda i: (0, 0)),
        ],
        out_specs=[
            pl.BlockSpec((RB3, DM), lambda i: (i, 0)),
            pl.BlockSpec((RB3, DM), lambda i: (i, 0)),
            pl.BlockSpec((2, DM), lambda i: (0, 0)),
        ],
        out_shape=[
            jax.ShapeDtypeStruct((RK, DM), jnp.float32),
            jax.ShapeDtypeStruct((RK, DM), jnp.float32),
            jax.ShapeDtypeStruct((2, DM), jnp.float32),
        ],
        interpret=_INTERPRET,
    )(t2, q_rows, kf, wg1, scale.reshape(1, DM), shift.reshape(1, DM))


def _a7_body(t4_ref, vf_ref, pe_ref, s_ref, t_ref, o_ref):
    a2 = _leaky(t4_ref[...] * s_ref[...] + t_ref[...])
    z = (a2 * (1.0 / np.sqrt(np.float32(N)))).reshape(RB3 // K, K, DM)
    m = jnp.max(z, axis=1, keepdims=True)
    e = jnp.exp(z - m)
    p = e / jnp.sum(e, axis=1, keepdims=True)
    w = (vf_ref[...] + pe_ref[...]).reshape(RB3 // K, K, DM)
    o_ref[...] = jnp.sum(p * w, axis=1)


def _stage_a7(t4, vf, pe, scale, shift):
    nb = RK // RB3
    return pl.pallas_call(
        _a7_body,
        grid=(nb,),
        in_specs=[
            pl.BlockSpec((RB3, DM), lambda i: (i, 0)),
            pl.BlockSpec((RB3, DM), lambda i: (i, 0)),
            pl.BlockSpec((RB3, DM), lambda i: (i, 0)),
            pl.BlockSpec((1, DM), lambda i: (0, 0)),
            pl.BlockSpec((1, DM), lambda i: (0, 0)),
        ],
        out_specs=pl.BlockSpec((RB3 // K, DM), lambda i: (i, 0)),
        out_shape=jax.ShapeDtypeStruct((R, DM), jnp.float32),
        interpret=_INTERPRET,
    )(t4, vf, pe, scale.reshape(1, DM), shift.reshape(1, DM))


def _a8_body(res_ref, w2_ref, o_ref, sums_ref):
    o = _dgt(res_ref[...], w2_ref[...])
    o_ref[...] = o
    _acc_sums(sums_ref, o, True)


def _stage_a8(res, w2):
    return pl.pallas_call(
        _a8_body,
        out_shape=[
            jax.ShapeDtypeStruct((R, DP), jnp.float32),
            jax.ShapeDtypeStruct((2, DP), jnp.float32),
        ],
        interpret=_INTERPRET,
    )(res, w2)


def _a9_body(t5_ref, feats_ref, s_ref, t_ref, o_ref):
    y = _leaky(t5_ref[...] * s_ref[...] + t_ref[...])
    o_ref[0] = y.T + feats_ref[0]


def _stage_a9(t5, feats, scale, shift):
    return pl.pallas_call(
        _a9_body,
        grid=(B,),
        in_specs=[
            pl.BlockSpec((N, DP), lambda b: (b, 0)),
            pl.BlockSpec((1, DP, N), lambda b: (b, 0, 0)),
            pl.BlockSpec((1, DP), lambda b: (0, 0)),
            pl.BlockSpec((1, DP), lambda b: (0, 0)),
        ],
        out_specs=pl.BlockSpec((1, DP, N), lambda b: (b, 0, 0)),
        out_shape=jax.ShapeDtypeStruct((B, DP, N), jnp.float32),
        interpret=_INTERPRET,
    )(t5, feats, scale.reshape(1, DP), shift.reshape(1, DP))


def kernel(feats, pos, params):
    p = params

    feats_rows = jnp.transpose(feats, (0, 2, 1)).reshape(R, DP)
    pos_t = jnp.transpose(pos, (0, 2, 1))
    pos16 = jnp.pad(pos_t, ((0, 0), (0, 0), (0, 13))).reshape(R, 16)
    wd1p = jnp.pad(p['Wd1'], ((0, 0), (0, 13)))

    d = _stage_a2(pos16, pos)
    idx_flat = _sc_topk(d)
    q_rows, k_rows, v_rows, wp = _stage_a1(feats_rows, pos16, wd1p, p)

    kf, vf, wpg = _sc_gather(k_rows, v_rows, wp, idx_flat)

    sums1 = _stage_a3(wp, wpg)
    s1, h1 = _affine(sums1, RK, p['gd1'], p['bd1'])
    t2, sums2 = _stage_a4(wp, wpg, p['Wd2'], s1, h1)
    s2, h2 = _affine(sums2, RK, p['gd2'], p['bd2'])
    pe, t3, sums3 = _stage_a5(t2, q_rows, kf, p['Wg1'], s2, h2)
    s3, h3 = _affine(sums3, RK, p['gg1'], p['bg1'])
    t4, sums4 = _mm_stage(t3, p['Wg2'], s3, h3)
    s4, h4 = _affine(sums4, RK, p['gg2'], p['bg2'])
    res = _stage_a7(t4, vf, pe, s4, h4)
    t5, sums5 = _stage_a8(res, p['W2'])
    s5, h5 = _affine(sums5, R, p['g2'], p['b2'])
    return _stage_a9(t5, feats, s5, h5)
